# baseline (speedup 1.0000x reference)
import functools

import jax
import jax.numpy as jnp
from jax import lax
from jax.experimental import pallas as pl
from jax.experimental.pallas import tpu as pltpu
from jax.experimental.pallas import tpu_sc as plsc

N = 10000
E = 320000
H = 5
D = 32
F = H * D
HP = 16
NC = 2
NS = 16
NW = NC * NS
C = 32
E_PAD = 327680
EPW = E_PAD // NW
NCHUNK = EPW // C
BLK = 8
NIDX = E_PAD // C
ACC_R = 10016
RPT = ACC_R // NS
VPR = F // 16
NB = NCHUNK // 16
_WB_SPLITS = tuple((r * 2 * C, 2 * C) for r in range(RPT // (2 * C))) + (
    ((RPT // (2 * C)) * 2 * C, RPT % (2 * C)),)
_SIGMA = [2 * j if j < 16 else 2 * j - 31 for j in range(32)]
_TAU = [0] * 32
for _j, _s in enumerate(_SIGMA):
    _TAU[_s] = _j
_PERM = [hh * 32 + _TAU[m] for hh in range(H) for m in range(32)]

_mesh = plsc.VectorSubcoreMesh(core_axis_name="c", subcore_axis_name="s",
                               num_cores=NC, num_subcores=NS)
_sc_params = pltpu.CompilerParams(use_tc_tiling_on_sc=False,
                                  needs_layout_passes=False)


def _mm1_body(x_ref, w_ref, al_ref, ar_ref, h_ref, el_ref, er_ref):
    h = jnp.dot(x_ref[...], w_ref[...], preferred_element_type=jnp.float32)
    h_ref[...] = h.astype(jnp.bfloat16)
    el_ref[...] = jnp.dot(h, al_ref[...], preferred_element_type=jnp.float32)
    er_ref[...] = jnp.dot(h, ar_ref[...], preferred_element_type=jnp.float32)


_mm1 = pl.pallas_call(
    _mm1_body,
    out_shape=(jax.ShapeDtypeStruct((N, F), jnp.bfloat16),
               jax.ShapeDtypeStruct((N, HP), jnp.float32),
               jax.ShapeDtypeStruct((N, HP), jnp.float32)))


def _combine(p_ref, dp_ref, b_ref, s_ref):
    invd = 1.0 / (dp_ref[0, :N] + dp_ref[1, :N] + 1e-9)
    scale = jnp.dot(invd, s_ref[...], preferred_element_type=jnp.float32)
    x = (p_ref[0, :N] + p_ref[1, :N]) * scale + b_ref[...]
    return jnp.where(x > 0.0, x, x * 0.01)


def _mm2_body(p_ref, dp_ref, b_ref, s_ref, w_ref, al_ref, ar_ref,
              h_ref, el_ref, er_ref):
    x = _combine(p_ref, dp_ref, b_ref, s_ref)
    h = jnp.dot(x, w_ref[...], preferred_element_type=jnp.float32)
    h_ref[...] = h.astype(jnp.bfloat16)
    el_ref[...] = jnp.dot(h, al_ref[...], preferred_element_type=jnp.float32)
    er_ref[...] = jnp.dot(h, ar_ref[...], preferred_element_type=jnp.float32)


_mm2 = pl.pallas_call(
    _mm2_body,
    out_shape=(jax.ShapeDtypeStruct((N, F), jnp.bfloat16),
               jax.ShapeDtypeStruct((N, HP), jnp.float32),
               jax.ShapeDtypeStruct((N, HP), jnp.float32)))


def _fin_body(p_ref, dp_ref, b_ref, s_ref, o_ref):
    o_ref[...] = _combine(p_ref, dp_ref, b_ref, s_ref)


_fin = pl.pallas_call(
    _fin_body, out_shape=jax.ShapeDtypeStruct((N, F), jnp.float32))


def _wait(src, dst, sem):
    pltpu.make_async_copy(src, dst, sem).wait()


def _edge_body(el_hbm, er_hbm, h_hbm, sidx_hbm, didx_hbm, dp_hbm, op_hbm,
               sblk, dblk, elg, erg, eebuf, hbuf, mbuf, dsh, acc,
               semb0, semb1, sel0, sel1, ser0, ser1,
               smh0, smh1, sema0, sema1, sms0, sms1):
    cid = lax.axis_index("c")
    sid = lax.axis_index("s")
    wid = sid * NC + cid
    row0 = wid * NCHUNK
    sel = (sel0, sel1)
    ser = (ser0, ser1)
    smh = (smh0, smh1)
    sema = (sema0, sema1)
    sms = (sms0, sms1)

    def zrow16(i, _):
        elg[i, :] = jnp.zeros((HP,), jnp.float32)
        return 0
    lax.fori_loop(0, 2 * C, zrow16, 0)

    def zrowf(i, _):
        for v in range(VPR):
            mbuf[i, pl.ds(16 * v, 16)] = jnp.zeros((16,), jnp.float32)
        return 0
    lax.fori_loop(0, 2 * C, zrowf, 0)
    off = sid * RPT
    for (o, sz) in _WB_SPLITS:
        pltpu.sync_copy(elg.at[pl.ds(0, sz)], dsh.at[pl.ds(off + o, sz)])
        pltpu.sync_copy(mbuf.at[pl.ds(0, sz)], acc.at[pl.ds(off + o, sz)])
    plsc.subcore_barrier()

    pltpu.sync_copy(sidx_hbm.at[pl.ds(row0, BLK)], sblk.at[0])
    pltpu.sync_copy(didx_hbm.at[pl.ds(row0, BLK)], dblk.at[0])
    pltpu.async_copy(sidx_hbm.at[pl.ds(row0 + BLK, BLK)], sblk.at[1], semb1)
    pltpu.async_copy(didx_hbm.at[pl.ds(row0 + BLK, BLK)], dblk.at[1], semb1)
    pltpu.async_copy(el_hbm.at[sblk.at[0, 0]], elg.at[pl.ds(0, C)], sel0)
    pltpu.async_copy(er_hbm.at[dblk.at[0, 0]], erg.at[pl.ds(0, C)], ser0)
    pltpu.async_copy(h_hbm.at[sblk.at[0, 0]], hbuf.at[pl.ds(0, C)], smh0)

    def block16(kk, _):
        for r in range(16):
            p = r & 1
            q = 1 - p
            i = kk * 16 + r
            pc = p * C
            qc = q * C
            if r >= 2:
                _wait(eebuf.at[pl.ds(pc, C)],
                      dsh.at[dblk.at[(r - 2) // 8, (r - 2) % 8]], sema[p])
                _wait(mbuf.at[pl.ds(pc, C)],
                      acc.at[dblk.at[(r - 2) // 8, (r - 2) % 8]], sms[p])
            else:

                @pl.when(kk > 0)
                def _():
                    _wait(eebuf.at[pl.ds(pc, C)],
                          dsh.at[dblk.at[1, 6 + r]], sema[p])
                    _wait(mbuf.at[pl.ds(pc, C)],
                          acc.at[dblk.at[1, 6 + r]], sms[p])
            if r == 1:

                @pl.when(kk > 0)
                def _():
                    pltpu.async_copy(
                        sidx_hbm.at[pl.ds(row0 + (2 * kk + 1) * BLK, BLK)],
                        sblk.at[1], semb1)
                    pltpu.async_copy(
                        didx_hbm.at[pl.ds(row0 + (2 * kk + 1) * BLK, BLK)],
                        dblk.at[1], semb1)
            if r == 9:

                @pl.when(kk < NB - 1)
                def _():
                    pltpu.async_copy(
                        sidx_hbm.at[pl.ds(row0 + (2 * kk + 2) * BLK, BLK)],
                        sblk.at[0], semb0)
                    pltpu.async_copy(
                        didx_hbm.at[pl.ds(row0 + (2 * kk + 2) * BLK, BLK)],
                        dblk.at[0], semb0)
            if r < 15:
                if r == 7:
                    _wait(sidx_hbm.at[pl.ds(0, BLK)], sblk.at[1], semb1)
                    _wait(didx_hbm.at[pl.ds(0, BLK)], dblk.at[1], semb1)
                b1 = (r + 1) // 8
                rr1 = (r + 1) % 8
                pltpu.async_copy(el_hbm.at[sblk.at[b1, rr1]],
                                 elg.at[pl.ds(qc, C)], sel[q])
                pltpu.async_copy(er_hbm.at[dblk.at[b1, rr1]],
                                 erg.at[pl.ds(qc, C)], ser[q])
                pltpu.async_copy(h_hbm.at[sblk.at[b1, rr1]],
                                 hbuf.at[pl.ds(qc, C)], smh[q])
            else:

                @pl.when(kk < NB - 1)
                def _():
                    _wait(sidx_hbm.at[pl.ds(0, BLK)], sblk.at[0], semb0)
                    _wait(didx_hbm.at[pl.ds(0, BLK)], dblk.at[0], semb0)
                    pltpu.async_copy(el_hbm.at[sblk.at[0, 0]],
                                     elg.at[pl.ds(qc, C)], sel[q])
                    pltpu.async_copy(er_hbm.at[dblk.at[0, 0]],
                                     erg.at[pl.ds(qc, C)], ser[q])
                    pltpu.async_copy(h_hbm.at[sblk.at[0, 0]],
                                     hbuf.at[pl.ds(qc, C)], smh[q])
            _wait(el_hbm.at[sblk.at[0, 0]], elg.at[pl.ds(pc, C)], sel[p])
            _wait(er_hbm.at[dblk.at[0, 0]], erg.at[pl.ds(pc, C)], ser[p])
            _wait(h_hbm.at[sblk.at[0, 0]], hbuf.at[pl.ds(pc, C)], smh[p])

            @plsc.parallel_loop(0, C, 1, unroll=4)
            def _(j):
                e = elg[pc + j, :] + erg[pc + j, :]
                e = jnp.where(e > 0.0, e, e * 0.2)
                a = jnp.exp(e)
                eebuf[pc + j, :] = a
                for v in range(H):
                    hv = hbuf[pc + j, pl.ds(32 * v, 32)]
                    lo, hi = plsc.unpack(hv,
                                         format=plsc.PackFormat.INTERLEAVED)
                    mbuf[pc + j, pl.ds(32 * v, 16)] = lo * a[v]
                    mbuf[pc + j, pl.ds(32 * v + 16, 16)] = hi * a[v]

            pltpu.async_copy(eebuf.at[pl.ds(pc, C)],
                             dsh.at[dblk.at[r // 8, r % 8]], sema[p],
                             add=True)
            pltpu.async_copy(mbuf.at[pl.ds(pc, C)],
                             acc.at[dblk.at[r // 8, r % 8]], sms[p],
                             add=True)
        return 0
    lax.fori_loop(0, NB, block16, 0)

    for p in (0, 1):
        _wait(eebuf.at[pl.ds(p * C, C)], dsh.at[dblk.at[1, 6 + p]], sema[p])
        _wait(mbuf.at[pl.ds(p * C, C)], acc.at[dblk.at[1, 6 + p]], sms[p])

    plsc.subcore_barrier()
    pltpu.sync_copy(dsh.at[pl.ds(off, RPT)], dp_hbm.at[cid, pl.ds(off, RPT)])
    pltpu.sync_copy(acc.at[pl.ds(off, RPT)], op_hbm.at[cid, pl.ds(off, RPT)])


_edge = pl.kernel(
    _edge_body,
    out_type=(jax.ShapeDtypeStruct((NC, ACC_R, HP), jnp.float32),
              jax.ShapeDtypeStruct((NC, ACC_R, F), jnp.float32)),
    mesh=_mesh,
    compiler_params=_sc_params,
    scratch_types=[
        pltpu.VMEM((2, BLK, C), jnp.int32),
        pltpu.VMEM((2, BLK, C), jnp.int32),
        pltpu.VMEM((2 * C, HP), jnp.float32),
        pltpu.VMEM((2 * C, HP), jnp.float32),
        pltpu.VMEM((2 * C, HP), jnp.float32),
        pltpu.VMEM((2 * C, F), jnp.bfloat16),
        pltpu.VMEM((2 * C, F), jnp.float32),
        pltpu.VMEM_SHARED((ACC_R, HP), jnp.float32),
        pltpu.VMEM_SHARED((ACC_R, F), jnp.float32),
        pltpu.SemaphoreType.DMA,
        pltpu.SemaphoreType.DMA,
        pltpu.SemaphoreType.DMA,
        pltpu.SemaphoreType.DMA,
        pltpu.SemaphoreType.DMA,
        pltpu.SemaphoreType.DMA,
        pltpu.SemaphoreType.DMA,
        pltpu.SemaphoreType.DMA,
        pltpu.SemaphoreType.DMA,
        pltpu.SemaphoreType.DMA,
        pltpu.SemaphoreType.DMA,
        pltpu.SemaphoreType.DMA,
    ])


def _attn_mat(a):
    A = jnp.zeros((F, HP), jnp.float32)
    return A.at[jnp.arange(F), jnp.repeat(jnp.arange(H), D)].set(
        a.astype(jnp.float32).reshape(F))


def _selector():
    S = jnp.zeros((HP, F), jnp.float32)
    return S.at[jnp.repeat(jnp.arange(H), D), jnp.arange(F)].set(1.0)


def kernel(x, edge_index, W1, al1, ar1, b1, W2, al2, ar2, b2,
           W3, al3, ar3, b3):
    perm = jnp.array(_PERM, jnp.int32)
    tau = jnp.array(_TAU, jnp.int32)
    W1, W2, W3 = W1[:, perm], W2[:, perm], W3[:, perm]
    al1, al2, al3 = al1[:, tau], al2[:, tau], al3[:, tau]
    ar1, ar2, ar3 = ar1[:, tau], ar2[:, tau], ar3[:, tau]
    src = jnp.asarray(edge_index[0], jnp.int32)
    dst = jnp.asarray(edge_index[1], jnp.int32)
    npad = E_PAD - E
    src2d = jnp.concatenate(
        [src, jnp.zeros((npad,), jnp.int32)]).reshape(NIDX, C)
    dst2d = jnp.concatenate(
        [dst, jnp.full((npad,), N, jnp.int32)]).reshape(NIDX, C)
    S = _selector()

    def edge_phase(h, el, er):
        dp, op = _edge(el, er, h, src2d, dst2d)
        return op, dp

    h, el, er = _mm1(x, W1, _attn_mat(al1), _attn_mat(ar1))
    op, dp = edge_phase(h, el, er)
    h, el, er = _mm2(op, dp, b1.reshape(1, F), S, W2,
                     _attn_mat(al2), _attn_mat(ar2))
    op, dp = edge_phase(h, el, er)
    h, el, er = _mm2(op, dp, b2.reshape(1, F), S, W3,
                     _attn_mat(al3), _attn_mat(ar3))
    op, dp = edge_phase(h, el, er)
    return _fin(op, dp, b3.reshape(1, F), S)


# speedup vs baseline: 1.0078x; 1.0078x over previous
import functools

import jax
import jax.numpy as jnp
from jax import lax
from jax.experimental import pallas as pl
from jax.experimental.pallas import tpu as pltpu
from jax.experimental.pallas import tpu_sc as plsc

N = 10000
E = 320000
H = 5
D = 32
F = H * D
HP = 16
NC = 2
NS = 16
NW = NC * NS
C = 32
E_PAD = 327680
EPW = E_PAD // NW
NCHUNK = EPW // C
BLK = 8
NIDX = E_PAD // C
ACC_R = 10016
RPT = ACC_R // NS
VPR = F // 16
NB = NCHUNK // 16
_WB_SPLITS = tuple((r * 2 * C, 2 * C) for r in range(RPT // (2 * C))) + (
    ((RPT // (2 * C)) * 2 * C, RPT % (2 * C)),)
_SIGMA = [2 * j if j < 16 else 2 * j - 31 for j in range(32)]
_TAU = [0] * 32
for _j, _s in enumerate(_SIGMA):
    _TAU[_s] = _j
_PERM = [hh * 32 + _TAU[m] for hh in range(H) for m in range(32)]

_mesh = plsc.VectorSubcoreMesh(core_axis_name="c", subcore_axis_name="s",
                               num_cores=NC, num_subcores=NS)
_sc_params = pltpu.CompilerParams(use_tc_tiling_on_sc=False,
                                  needs_layout_passes=False)


def _mm1_body(x_ref, w_ref, al_ref, ar_ref, h_ref, el_ref, er_ref):
    h = jnp.dot(x_ref[...], w_ref[...], preferred_element_type=jnp.float32)
    h_ref[...] = h.astype(jnp.bfloat16)
    el_ref[...] = jnp.dot(h, al_ref[...], preferred_element_type=jnp.float32)
    er_ref[...] = jnp.dot(h, ar_ref[...], preferred_element_type=jnp.float32)


_mm1 = pl.pallas_call(
    _mm1_body,
    out_shape=(jax.ShapeDtypeStruct((N, F), jnp.bfloat16),
               jax.ShapeDtypeStruct((N, HP), jnp.float32),
               jax.ShapeDtypeStruct((N, HP), jnp.float32)))


def _combine(p_ref, dp_ref, b_ref, s_ref):
    invd = 1.0 / (dp_ref[0, :N] + dp_ref[1, :N] + 1e-9)
    scale = jnp.dot(invd, s_ref[...], preferred_element_type=jnp.float32)
    x = (p_ref[0, :N] + p_ref[1, :N]) * scale + b_ref[...]
    return jnp.where(x > 0.0, x, x * 0.01)


def _mm2_body(p_ref, dp_ref, b_ref, s_ref, w_ref, al_ref, ar_ref,
              h_ref, el_ref, er_ref):
    x = _combine(p_ref, dp_ref, b_ref, s_ref)
    h = jnp.dot(x, w_ref[...], preferred_element_type=jnp.float32)
    h_ref[...] = h.astype(jnp.bfloat16)
    el_ref[...] = jnp.dot(h, al_ref[...], preferred_element_type=jnp.float32)
    er_ref[...] = jnp.dot(h, ar_ref[...], preferred_element_type=jnp.float32)


_mm2 = pl.pallas_call(
    _mm2_body,
    out_shape=(jax.ShapeDtypeStruct((N, F), jnp.bfloat16),
               jax.ShapeDtypeStruct((N, HP), jnp.float32),
               jax.ShapeDtypeStruct((N, HP), jnp.float32)))


def _fin_body(p_ref, dp_ref, b_ref, s_ref, o_ref):
    o_ref[...] = _combine(p_ref, dp_ref, b_ref, s_ref)


_fin = pl.pallas_call(
    _fin_body, out_shape=jax.ShapeDtypeStruct((N, F), jnp.float32))


def _wait(src, dst, sem):
    pltpu.make_async_copy(src, dst, sem).wait()


def _edge_body(el_hbm, er_hbm, h_hbm, sidx_hbm, didx_hbm, dp_hbm, op_hbm,
               sblk, dblk, elg, erg, eebuf, hbuf, mbuf, dsh, acc,
               semb0, semb1, sel0, sel1, ser0, ser1,
               smh0, smh1, sema0, sema1, sms0, sms1):
    cid = lax.axis_index("c")
    sid = lax.axis_index("s")
    wid = sid * NC + cid
    row0 = wid * NCHUNK
    sel = (sel0, sel1)
    ser = (ser0, ser1)
    smh = (smh0, smh1)
    sema = (sema0, sema1)
    sms = (sms0, sms1)

    def zrow16(i, _):
        elg[i, :] = jnp.zeros((HP,), jnp.float32)
        return 0
    lax.fori_loop(0, 2 * C, zrow16, 0)

    def zrowf(i, _):
        for v in range(VPR):
            mbuf[i, pl.ds(16 * v, 16)] = jnp.zeros((16,), jnp.float32)
        return 0
    lax.fori_loop(0, 2 * C, zrowf, 0)
    off = sid * RPT
    for (o, sz) in _WB_SPLITS:
        pltpu.sync_copy(elg.at[pl.ds(0, sz)], dsh.at[pl.ds(off + o, sz)])
        pltpu.sync_copy(mbuf.at[pl.ds(0, sz)], acc.at[pl.ds(off + o, sz)])
    plsc.subcore_barrier()

    pltpu.sync_copy(sidx_hbm.at[pl.ds(row0, BLK)], sblk.at[0])
    pltpu.sync_copy(didx_hbm.at[pl.ds(row0, BLK)], dblk.at[0])
    pltpu.async_copy(sidx_hbm.at[pl.ds(row0 + BLK, BLK)], sblk.at[1], semb1)
    pltpu.async_copy(didx_hbm.at[pl.ds(row0 + BLK, BLK)], dblk.at[1], semb1)
    pltpu.async_copy(el_hbm.at[sblk.at[0, 0]], elg.at[pl.ds(0, C)], sel0)
    pltpu.async_copy(er_hbm.at[dblk.at[0, 0]], erg.at[pl.ds(0, C)], ser0)
    pltpu.async_copy(h_hbm.at[sblk.at[0, 0]], hbuf.at[pl.ds(0, C)], smh0)

    def block16(kk, _):
        for r in range(16):
            p = r & 1
            q = 1 - p
            i = kk * 16 + r
            pc = p * C
            qc = q * C
            if r >= 2:
                _wait(eebuf.at[pl.ds(pc, C)],
                      dsh.at[dblk.at[(r - 2) // 8, (r - 2) % 8]], sema[p])
                _wait(mbuf.at[pl.ds(pc, C)],
                      acc.at[dblk.at[(r - 2) // 8, (r - 2) % 8]], sms[p])
            else:

                @pl.when(kk > 0)
                def _():
                    _wait(eebuf.at[pl.ds(pc, C)],
                          dsh.at[dblk.at[1, 6 + r]], sema[p])
                    _wait(mbuf.at[pl.ds(pc, C)],
                          acc.at[dblk.at[1, 6 + r]], sms[p])
            if r == 1:

                @pl.when(kk > 0)
                def _():
                    pltpu.async_copy(
                        sidx_hbm.at[pl.ds(row0 + (2 * kk + 1) * BLK, BLK)],
                        sblk.at[1], semb1)
                    pltpu.async_copy(
                        didx_hbm.at[pl.ds(row0 + (2 * kk + 1) * BLK, BLK)],
                        dblk.at[1], semb1)
            if r == 9:

                @pl.when(kk < NB - 1)
                def _():
                    pltpu.async_copy(
                        sidx_hbm.at[pl.ds(row0 + (2 * kk + 2) * BLK, BLK)],
                        sblk.at[0], semb0)
                    pltpu.async_copy(
                        didx_hbm.at[pl.ds(row0 + (2 * kk + 2) * BLK, BLK)],
                        dblk.at[0], semb0)
            if r < 15:
                if r == 7:
                    _wait(sidx_hbm.at[pl.ds(0, BLK)], sblk.at[1], semb1)
                    _wait(didx_hbm.at[pl.ds(0, BLK)], dblk.at[1], semb1)
                b1 = (r + 1) // 8
                rr1 = (r + 1) % 8
                pltpu.async_copy(el_hbm.at[sblk.at[b1, rr1]],
                                 elg.at[pl.ds(qc, C)], sel[q])
                pltpu.async_copy(er_hbm.at[dblk.at[b1, rr1]],
                                 erg.at[pl.ds(qc, C)], ser[q])
                pltpu.async_copy(h_hbm.at[sblk.at[b1, rr1]],
                                 hbuf.at[pl.ds(qc, C)], smh[q])
            else:

                @pl.when(kk < NB - 1)
                def _():
                    _wait(sidx_hbm.at[pl.ds(0, BLK)], sblk.at[0], semb0)
                    _wait(didx_hbm.at[pl.ds(0, BLK)], dblk.at[0], semb0)
                    pltpu.async_copy(el_hbm.at[sblk.at[0, 0]],
                                     elg.at[pl.ds(qc, C)], sel[q])
                    pltpu.async_copy(er_hbm.at[dblk.at[0, 0]],
                                     erg.at[pl.ds(qc, C)], ser[q])
                    pltpu.async_copy(h_hbm.at[sblk.at[0, 0]],
                                     hbuf.at[pl.ds(qc, C)], smh[q])
            _wait(el_hbm.at[sblk.at[0, 0]], elg.at[pl.ds(pc, C)], sel[p])
            _wait(er_hbm.at[dblk.at[0, 0]], erg.at[pl.ds(pc, C)], ser[p])
            _wait(h_hbm.at[sblk.at[0, 0]], hbuf.at[pl.ds(pc, C)], smh[p])

            @plsc.parallel_loop(0, C, 1, unroll=2)
            def _(j):
                e = elg[pc + j, :] + erg[pc + j, :]
                e = jnp.where(e > 0.0, e, e * 0.2)
                a = jnp.exp(e)
                eebuf[pc + j, :] = a
                for v in range(H):
                    hv = hbuf[pc + j, pl.ds(32 * v, 32)]
                    lo, hi = plsc.unpack(hv,
                                         format=plsc.PackFormat.INTERLEAVED)
                    mbuf[pc + j, pl.ds(32 * v, 16)] = lo * a[v]
                    mbuf[pc + j, pl.ds(32 * v + 16, 16)] = hi * a[v]

            pltpu.async_copy(eebuf.at[pl.ds(pc, C)],
                             dsh.at[dblk.at[r // 8, r % 8]], sema[p],
                             add=True)
            pltpu.async_copy(mbuf.at[pl.ds(pc, C)],
                             acc.at[dblk.at[r // 8, r % 8]], sms[p],
                             add=True)
        return 0
    lax.fori_loop(0, NB, block16, 0)

    for p in (0, 1):
        _wait(eebuf.at[pl.ds(p * C, C)], dsh.at[dblk.at[1, 6 + p]], sema[p])
        _wait(mbuf.at[pl.ds(p * C, C)], acc.at[dblk.at[1, 6 + p]], sms[p])

    plsc.subcore_barrier()
    for (o, sz) in _WB_SPLITS:
        pltpu.sync_copy(dsh.at[pl.ds(off + o, sz)], elg.at[pl.ds(0, sz)])
        pltpu.sync_copy(elg.at[pl.ds(0, sz)],
                        dp_hbm.at[cid, pl.ds(off + o, sz)])
        pltpu.sync_copy(acc.at[pl.ds(off + o, sz)], mbuf.at[pl.ds(0, sz)])
        pltpu.sync_copy(mbuf.at[pl.ds(0, sz)],
                        op_hbm.at[cid, pl.ds(off + o, sz)])


_edge = pl.kernel(
    _edge_body,
    out_type=(jax.ShapeDtypeStruct((NC, ACC_R, HP), jnp.float32),
              jax.ShapeDtypeStruct((NC, ACC_R, F), jnp.float32)),
    mesh=_mesh,
    compiler_params=_sc_params,
    scratch_types=[
        pltpu.VMEM((2, BLK, C), jnp.int32),
        pltpu.VMEM((2, BLK, C), jnp.int32),
        pltpu.VMEM((2 * C, HP), jnp.float32),
        pltpu.VMEM((2 * C, HP), jnp.float32),
        pltpu.VMEM((2 * C, HP), jnp.float32),
        pltpu.VMEM((2 * C, F), jnp.bfloat16),
        pltpu.VMEM((2 * C, F), jnp.float32),
        pltpu.VMEM_SHARED((ACC_R, HP), jnp.float32),
        pltpu.VMEM_SHARED((ACC_R, F), jnp.float32),
        pltpu.SemaphoreType.DMA,
        pltpu.SemaphoreType.DMA,
        pltpu.SemaphoreType.DMA,
        pltpu.SemaphoreType.DMA,
        pltpu.SemaphoreType.DMA,
        pltpu.SemaphoreType.DMA,
        pltpu.SemaphoreType.DMA,
        pltpu.SemaphoreType.DMA,
        pltpu.SemaphoreType.DMA,
        pltpu.SemaphoreType.DMA,
        pltpu.SemaphoreType.DMA,
        pltpu.SemaphoreType.DMA,
    ])


def _attn_mat(a):
    A = jnp.zeros((F, HP), jnp.float32)
    return A.at[jnp.arange(F), jnp.repeat(jnp.arange(H), D)].set(
        a.astype(jnp.float32).reshape(F))


def _selector():
    S = jnp.zeros((HP, F), jnp.float32)
    return S.at[jnp.repeat(jnp.arange(H), D), jnp.arange(F)].set(1.0)


def kernel(x, edge_index, W1, al1, ar1, b1, W2, al2, ar2, b2,
           W3, al3, ar3, b3):
    perm = jnp.array(_PERM, jnp.int32)
    tau = jnp.array(_TAU, jnp.int32)
    W1, W2, W3 = W1[:, perm], W2[:, perm], W3[:, perm]
    al1, al2, al3 = al1[:, tau], al2[:, tau], al3[:, tau]
    ar1, ar2, ar3 = ar1[:, tau], ar2[:, tau], ar3[:, tau]
    src = jnp.asarray(edge_index[0], jnp.int32)
    dst = jnp.asarray(edge_index[1], jnp.int32)
    npad = E_PAD - E
    src2d = jnp.concatenate(
        [src, jnp.zeros((npad,), jnp.int32)]).reshape(NIDX, C)
    dst2d = jnp.concatenate(
        [dst, jnp.full((npad,), N, jnp.int32)]).reshape(NIDX, C)
    S = _selector()

    def edge_phase(h, el, er):
        dp, op = _edge(el, er, h, src2d, dst2d)
        return op, dp

    h, el, er = _mm1(x, W1, _attn_mat(al1), _attn_mat(ar1))
    op, dp = edge_phase(h, el, er)
    h, el, er = _mm2(op, dp, b1.reshape(1, F), S, W2,
                     _attn_mat(al2), _attn_mat(ar2))
    op, dp = edge_phase(h, el, er)
    h, el, er = _mm2(op, dp, b2.reshape(1, F), S, W3,
                     _attn_mat(al3), _attn_mat(ar3))
    op, dp = edge_phase(h, el, er)
    return _fin(op, dp, b3.reshape(1, F), S)
